# Initial kernel scaffold; baseline (speedup 1.0000x reference)
import jax
import jax.numpy as jnp
from jax.experimental import pallas as pl


def kernel(x, edge_index, batch, W1, b1, W2, b2, W3, b3, lin_W, lin_b):
    raise NotImplementedError("write your pallas kernel here")


# speedup vs baseline: 12.3067x; 12.3067x over previous
import functools

import jax
import jax.numpy as jnp
from jax import lax
from jax.experimental import pallas as pl
from jax.experimental.pallas import tpu as pltpu
from jax.experimental.pallas import tpu_sc as plsc

_f32 = jnp.float32
_i32 = jnp.int32

NC = 2
NS = 16
NW = NC * NS

N = 10000
E = 320000
D = 128
G = 600

NPAD = 10240
GPAD = 1024
CHUNK = 80


def _sc_mesh():
    return plsc.VectorSubcoreMesh(
        core_axis_name="c", subcore_axis_name="s",
        num_cores=NC, num_subcores=NS)


def _fill_vec(ref, n, value):
    def body(i, _):
        ref[pl.ds(i * 16, 16)] = jnp.full((16,), value, _f32)
        return 0
    lax.fori_loop(0, n // 16, body, 0)


def _zero_rows(ref, rows, k):
    kv = k // 16
    def body(i, _):
        r = i // kv
        j = i % kv
        ref[r, pl.ds(j * 16, 16)] = jnp.zeros((16,), _f32)
        return 0
    lax.fori_loop(0, rows * kv, body, 0)


def _make_deg_kernel():
    stripe = NPAD // NS
    e_per_tile = E // NW
    nchunks = e_per_tile // CHUNK

    @functools.partial(
        pl.kernel,
        out_type=jax.ShapeDtypeStruct((NC, NS, stripe), _f32),
        mesh=_sc_mesh(),
        scratch_types=[
            pltpu.VMEM((CHUNK,), _i32),
            pltpu.VMEM((CHUNK,), _f32),
            pltpu.VMEM((stripe,), _f32),
            pltpu.VMEM_SHARED((NPAD,), _f32),
            pltpu.SemaphoreType.DMA,
        ],
        compiler_params=pltpu.CompilerParams(use_tc_tiling_on_sc=False),
    )
    def deg_kernel(dst_hbm, out_hbm, idx_v, ones_v, zero_v, acc_sh, sem):
        c = lax.axis_index("c")
        s = lax.axis_index("s")
        wid = s * NC + c
        _fill_vec(ones_v, CHUNK, 1.0)
        _fill_vec(zero_v, stripe, 0.0)
        pltpu.sync_copy(zero_v, acc_sh.at[pl.ds(s * stripe, stripe)])
        plsc.subcore_barrier()
        base = wid * e_per_tile

        def body(i, _):
            pltpu.sync_copy(dst_hbm.at[pl.ds(base + i * CHUNK, CHUNK)], idx_v)
            pltpu.sync_copy(ones_v, acc_sh.at[idx_v], add=True)
            return 0
        lax.fori_loop(0, nchunks, body, 0)
        plsc.subcore_barrier()
        pltpu.sync_copy(acc_sh.at[pl.ds(s * stripe, stripe)], out_hbm.at[c, s])

    return deg_kernel


def _make_edge_kernel(k):
    stripe = NPAD // NS
    e_per_tile = E // NW
    nchunks = e_per_tile // CHUNK
    nzero = stripe // CHUNK

    @functools.partial(
        pl.kernel,
        out_type=jax.ShapeDtypeStruct((NC, NS, stripe, k), _f32),
        mesh=_sc_mesh(),
        scratch_types=[
            pltpu.VMEM((CHUNK,), _i32),
            pltpu.VMEM((CHUNK,), _i32),
            pltpu.VMEM((CHUNK, k), _f32),
            pltpu.VMEM_SHARED((NPAD, k), _f32),
            pltpu.SemaphoreType.DMA,
        ],
        compiler_params=pltpu.CompilerParams(use_tc_tiling_on_sc=False),
    )
    def edge_kernel(table_hbm, src_hbm, dst_hbm, out_hbm,
                    src_v, dst_v, rows_v, acc_sh, sem):
        c = lax.axis_index("c")
        s = lax.axis_index("s")
        wid = s * NC + c
        _zero_rows(rows_v, CHUNK, k)

        def zc(i, _):
            pltpu.sync_copy(
                rows_v, acc_sh.at[pl.ds(s * stripe + i * CHUNK, CHUNK)])
            return 0
        lax.fori_loop(0, nzero, zc, 0)
        plsc.subcore_barrier()
        base = wid * e_per_tile

        def body(i, _):
            off = base + i * CHUNK
            pltpu.sync_copy(src_hbm.at[pl.ds(off, CHUNK)], src_v)
            pltpu.sync_copy(dst_hbm.at[pl.ds(off, CHUNK)], dst_v)
            pltpu.async_copy(table_hbm.at[src_v], rows_v, sem).wait()
            pltpu.sync_copy(rows_v, acc_sh.at[dst_v], add=True)
            return 0
        lax.fori_loop(0, nchunks, body, 0)
        plsc.subcore_barrier()
        pltpu.sync_copy(acc_sh.at[pl.ds(s * stripe, stripe)], out_hbm.at[c, s])

    return edge_kernel


def _make_pool_kernel(k):
    gstripe = GPAD // NS
    nchunks_total = N // CHUNK

    @functools.partial(
        pl.kernel,
        out_type=(jax.ShapeDtypeStruct((NC, NS, gstripe, k), _f32),
                  jax.ShapeDtypeStruct((NC, NS, gstripe), _f32)),
        mesh=_sc_mesh(),
        scratch_types=[
            pltpu.VMEM((CHUNK,), _i32),
            pltpu.VMEM((CHUNK,), _f32),
            pltpu.VMEM((CHUNK, k), _f32),
            pltpu.VMEM((gstripe,), _f32),
            pltpu.VMEM_SHARED((GPAD, k), _f32),
            pltpu.VMEM_SHARED((GPAD,), _f32),
            pltpu.SemaphoreType.DMA,
        ],
        compiler_params=pltpu.CompilerParams(use_tc_tiling_on_sc=False),
    )
    def pool_kernel(h_hbm, batch_hbm, sums_hbm, cnts_hbm,
                    idx_v, ones_v, rows_v, zero_v, sums_sh, cnts_sh, sem):
        c = lax.axis_index("c")
        s = lax.axis_index("s")
        wid = s * NC + c
        _fill_vec(ones_v, CHUNK, 1.0)
        _fill_vec(zero_v, gstripe, 0.0)
        _zero_rows(rows_v, CHUNK, k)
        pltpu.sync_copy(rows_v.at[pl.ds(0, gstripe)],
                        sums_sh.at[pl.ds(s * gstripe, gstripe)])
        pltpu.sync_copy(zero_v, cnts_sh.at[pl.ds(s * gstripe, gstripe)])
        plsc.subcore_barrier()
        my_chunks = (nchunks_total - wid + NW - 1) // NW

        def body(i, _):
            off = (wid + i * NW) * CHUNK
            pltpu.sync_copy(batch_hbm.at[pl.ds(off, CHUNK)], idx_v)
            pltpu.sync_copy(h_hbm.at[pl.ds(off, CHUNK)], rows_v)
            pltpu.sync_copy(rows_v, sums_sh.at[idx_v], add=True)
            pltpu.sync_copy(ones_v, cnts_sh.at[idx_v], add=True)
            return 0
        lax.fori_loop(0, my_chunks, body, 0)
        plsc.subcore_barrier()
        pltpu.sync_copy(sums_sh.at[pl.ds(s * gstripe, gstripe)],
                        sums_hbm.at[c, s])
        pltpu.sync_copy(cnts_sh.at[pl.ds(s * gstripe, gstripe)],
                        cnts_hbm.at[c, s])

    return pool_kernel


_TCB = 2000


def _tc_first(deg_cols, x, w1):
    d = x.shape[1]
    k = w1.shape[1]

    def body(deg_ref, x_ref, w_ref, dinv_ref, hw_ref):
        deg = deg_ref[:, 0:1] + deg_ref[:, 1:2] + 1.0
        dinv = lax.rsqrt(deg)
        hw = jnp.dot(x_ref[...], w_ref[...], preferred_element_type=_f32)
        dinv_ref[...] = dinv
        hw_ref[...] = hw * dinv

    return pl.pallas_call(
        body,
        grid=(N // _TCB,),
        in_specs=[
            pl.BlockSpec((_TCB, NC), lambda i: (i, 0)),
            pl.BlockSpec((_TCB, d), lambda i: (i, 0)),
            pl.BlockSpec((d, k), lambda i: (0, 0)),
        ],
        out_specs=[
            pl.BlockSpec((_TCB, 1), lambda i: (i, 0)),
            pl.BlockSpec((_TCB, k), lambda i: (i, 0)),
        ],
        out_shape=[
            jax.ShapeDtypeStruct((N, 1), _f32),
            jax.ShapeDtypeStruct((N, k), _f32),
        ],
    )(deg_cols, x, w1)


def _tc_mid(acc, hwp, dinv, b, w_next):
    k = hwp.shape[1]
    k2 = w_next.shape[1]

    def body(acc_ref, hwp_ref, dinv_ref, b_ref, w_ref, out_ref):
        t = acc_ref[0] + acc_ref[1] + hwp_ref[...]
        h = jnp.maximum(t * dinv_ref[...] + b_ref[...], 0.0)
        out_ref[...] = (
            jnp.dot(h, w_ref[...], preferred_element_type=_f32)
            * dinv_ref[...])

    return pl.pallas_call(
        body,
        grid=(N // _TCB,),
        in_specs=[
            pl.BlockSpec((NC, _TCB, k), lambda i: (0, i, 0)),
            pl.BlockSpec((_TCB, k), lambda i: (i, 0)),
            pl.BlockSpec((_TCB, 1), lambda i: (i, 0)),
            pl.BlockSpec((1, k), lambda i: (0, 0)),
            pl.BlockSpec((k, k2), lambda i: (0, 0)),
        ],
        out_specs=pl.BlockSpec((_TCB, k2), lambda i: (i, 0)),
        out_shape=jax.ShapeDtypeStruct((N, k2), _f32),
    )(acc, hwp, dinv, b, w_next)


def _tc_last_h(acc, hwp, dinv, b):
    k = hwp.shape[1]

    def body(acc_ref, hwp_ref, dinv_ref, b_ref, out_ref):
        t = acc_ref[0] + acc_ref[1] + hwp_ref[...]
        out_ref[...] = jnp.maximum(t * dinv_ref[...] + b_ref[...], 0.0)

    return pl.pallas_call(
        body,
        grid=(N // _TCB,),
        in_specs=[
            pl.BlockSpec((NC, _TCB, k), lambda i: (0, i, 0)),
            pl.BlockSpec((_TCB, k), lambda i: (i, 0)),
            pl.BlockSpec((_TCB, 1), lambda i: (i, 0)),
            pl.BlockSpec((1, k), lambda i: (0, 0)),
        ],
        out_specs=pl.BlockSpec((_TCB, k), lambda i: (i, 0)),
        out_shape=jax.ShapeDtypeStruct((N, k), _f32),
    )(acc, hwp, dinv, b)


def _tc_head(sums, cnts_cols, lin_w, lin_b):
    k = sums.shape[2]
    ncls = lin_w.shape[1]

    def body(s_ref, c_ref, w_ref, b_ref, out_ref):
        cnt = c_ref[:, 0:1] + c_ref[:, 1:2]
        pooled = (s_ref[0] + s_ref[1]) / jnp.maximum(cnt, 1.0)
        out_ref[...] = (
            jnp.dot(pooled, w_ref[...], preferred_element_type=_f32)
            + b_ref[...])

    return pl.pallas_call(
        body,
        grid=(1,),
        in_specs=[
            pl.BlockSpec((NC, G, k), lambda i: (0, 0, 0)),
            pl.BlockSpec((G, NC), lambda i: (0, 0)),
            pl.BlockSpec((k, ncls), lambda i: (0, 0)),
            pl.BlockSpec((1, ncls), lambda i: (0, 0)),
        ],
        out_specs=pl.BlockSpec((G, ncls), lambda i: (0, 0)),
        out_shape=jax.ShapeDtypeStruct((G, ncls), _f32),
    )(sums, cnts_cols, lin_w, lin_b)


_deg_kernel = _make_deg_kernel()
_edge64 = _make_edge_kernel(64)
_edge128 = _make_edge_kernel(128)
_pool64 = _make_pool_kernel(64)


def kernel(x, edge_index, batch, W1, b1, W2, b2, W3, b3, lin_W, lin_b):
    src = edge_index[0]
    dst = edge_index[1]

    deg_p = _deg_kernel(dst)
    deg_cols = deg_p.reshape(NC, NPAD).T
    dinv, hw1p = _tc_first(deg_cols, x, W1)

    acc1 = _edge64(hw1p, src, dst).reshape(NC, NPAD, 64)
    hw2p = _tc_mid(acc1, hw1p, dinv, b1.reshape(1, -1), W2)

    acc2 = _edge128(hw2p, src, dst).reshape(NC, NPAD, 128)
    hw3p = _tc_mid(acc2, hw2p, dinv, b2.reshape(1, -1), W3)

    acc3 = _edge64(hw3p, src, dst).reshape(NC, NPAD, 64)
    h3 = _tc_last_h(acc3, hw3p, dinv, b3.reshape(1, -1))

    sums, cnts = _pool64(h3, batch)
    out = _tc_head(sums.reshape(NC, GPAD, 64),
                   cnts.reshape(NC, GPAD).T,
                   lin_W, lin_b.reshape(1, -1))
    return out


# speedup vs baseline: 33.1498x; 2.6936x over previous
import functools

import jax
import jax.numpy as jnp
from jax import lax
from jax.experimental import pallas as pl
from jax.experimental.pallas import tpu as pltpu
from jax.experimental.pallas import tpu_sc as plsc

_f32 = jnp.float32
_i32 = jnp.int32

NC = 2
NS = 16
NW = NC * NS

N = 10000
E = 320000
D = 128
G = 600

NPAD = 10240
GPAD = 1024
CHUNK = 80
NCHUNKS_E = E // NW // CHUNK
NCHUNKS_N = N // CHUNK
NBUF = 5
STRIPE = NPAD // NS
GSTRIPE = GPAD // NS


def _sc_mesh():
    return plsc.VectorSubcoreMesh(
        core_axis_name="c", subcore_axis_name="s",
        num_cores=NC, num_subcores=NS)


def _make_deg_kernel():
    @functools.partial(
        pl.kernel,
        out_type=jax.ShapeDtypeStruct((NC, NS, STRIPE), _f32),
        mesh=_sc_mesh(),
        scratch_types=[
            pltpu.VMEM((NCHUNKS_E, CHUNK), _i32),
            pltpu.VMEM((CHUNK,), _f32),
            pltpu.VMEM_SHARED((NPAD,), _f32),
            pltpu.SemaphoreType.DMA,
            pltpu.SemaphoreType.DMA,
        ],
        compiler_params=pltpu.CompilerParams(use_tc_tiling_on_sc=False),
    )
    def deg_kernel(dst3_hbm, ones_hbm, zeros_hbm, out_hbm,
                   dst_all, ones_v, acc_sh, isem, ssem):
        c = lax.axis_index("c")
        s = lax.axis_index("s")
        wid = s * NC + c
        pltpu.async_copy(dst3_hbm.at[wid], dst_all, isem)
        pltpu.async_copy(ones_hbm, ones_v, isem)
        pltpu.async_copy(zeros_hbm, acc_sh.at[pl.ds(s * STRIPE, STRIPE)], isem)
        pltpu.make_async_copy(dst3_hbm.at[0], dst_all, isem).wait()
        pltpu.make_async_copy(ones_hbm, ones_v, isem).wait()
        pltpu.make_async_copy(
            zeros_hbm, acc_sh.at[pl.ds(s * STRIPE, STRIPE)], isem).wait()
        plsc.subcore_barrier()

        def fire(i, _):
            pltpu.async_copy(ones_v, acc_sh.at[dst_all.at[i]], ssem, add=True)
            return 0
        lax.fori_loop(0, NCHUNKS_E, fire, 0)

        def drain(i, _):
            pltpu.make_async_copy(ones_hbm, ones_v, ssem).wait()
            return 0
        lax.fori_loop(0, NCHUNKS_E, drain, 0)
        plsc.subcore_barrier()
        pltpu.sync_copy(acc_sh.at[pl.ds(s * STRIPE, STRIPE)], out_hbm.at[c, s])

    return deg_kernel


def _make_edge_kernel(k, chunk):
    nchunks = E // NW // chunk
    ngroups = nchunks // NBUF

    @functools.partial(
        pl.kernel,
        out_type=jax.ShapeDtypeStruct((NC, NS, STRIPE, k), _f32),
        mesh=_sc_mesh(),
        scratch_types=(
            [pltpu.VMEM((nchunks, chunk), _i32),
             pltpu.VMEM((nchunks, chunk), _i32),
             pltpu.VMEM((NBUF, chunk, k), _f32),
             pltpu.VMEM_SHARED((NPAD, k), _f32),
             pltpu.SemaphoreType.DMA]
            + [pltpu.SemaphoreType.DMA] * NBUF
            + [pltpu.SemaphoreType.DMA] * NBUF
        ),
        compiler_params=pltpu.CompilerParams(use_tc_tiling_on_sc=False),
    )
    def edge_kernel(table_hbm, src3_hbm, dst3_hbm, zeros_hbm, out_hbm,
                    src_all, dst_all, rings, acc_sh, isem, *sems):
        gsems = sems[:NBUF]
        ssems = sems[NBUF:]
        c = lax.axis_index("c")
        s = lax.axis_index("s")
        wid = s * NC + c
        pltpu.async_copy(src3_hbm.at[wid], src_all, isem)
        pltpu.async_copy(dst3_hbm.at[wid], dst_all, isem)
        pltpu.async_copy(zeros_hbm, acc_sh.at[pl.ds(s * STRIPE, STRIPE)], isem)
        pltpu.make_async_copy(src3_hbm.at[0], src_all, isem).wait()
        pltpu.make_async_copy(dst3_hbm.at[0], dst_all, isem).wait()
        pltpu.make_async_copy(
            zeros_hbm, acc_sh.at[pl.ds(s * STRIPE, STRIPE)], isem).wait()
        plsc.subcore_barrier()

        def gather(i, b, sem):
            pltpu.async_copy(table_hbm.at[src_all.at[i]], rings.at[b], sem)

        def wait_rowchunk(sem, b):
            pltpu.make_async_copy(
                table_hbm.at[pl.ds(0, chunk)], rings.at[b], sem).wait()

        for b in range(NBUF):
            gather(b, b, gsems[b])

        def group(g, _):
            ibase = g * NBUF
            for b in range(NBUF):
                wait_rowchunk(gsems[b], b)
                pltpu.async_copy(
                    rings.at[b], acc_sh.at[dst_all.at[ibase + b]],
                    ssems[b], add=True)
            for b in range(NBUF):
                def refill(b=b, g=g):
                    wait_rowchunk(ssems[b], b)
                    gather((g + 1) * NBUF + b, b, gsems[b])
                pl.when(g + 1 < ngroups)(refill)
            return 0
        lax.fori_loop(0, ngroups, group, 0)
        for b in range(NBUF):
            wait_rowchunk(ssems[b], b)
        plsc.subcore_barrier()
        pltpu.sync_copy(acc_sh.at[pl.ds(s * STRIPE, STRIPE)], out_hbm.at[c, s])

    return edge_kernel


def _make_pool_kernel(k):
    maxc = (NCHUNKS_N + NW - 1) // NW

    @functools.partial(
        pl.kernel,
        out_type=(jax.ShapeDtypeStruct((NC, NS, GSTRIPE, k), _f32),
                  jax.ShapeDtypeStruct((NC, NS, GSTRIPE), _f32)),
        mesh=_sc_mesh(),
        scratch_types=[
            pltpu.VMEM((maxc, CHUNK), _i32),
            pltpu.VMEM((CHUNK,), _f32),
            pltpu.VMEM((maxc, CHUNK, k), _f32),
            pltpu.VMEM_SHARED((GPAD, k), _f32),
            pltpu.VMEM_SHARED((GPAD,), _f32),
            pltpu.SemaphoreType.DMA,
            pltpu.SemaphoreType.DMA,
        ],
        compiler_params=pltpu.CompilerParams(use_tc_tiling_on_sc=False),
    )
    def pool_kernel(h_hbm, batch3_hbm, ones_hbm, zeros_hbm,
                    sums_hbm, cnts_hbm,
                    idx_all, ones_v, rings, sums_sh, cnts_sh, isem, ssem):
        c = lax.axis_index("c")
        s = lax.axis_index("s")
        wid = s * NC + c
        my_chunks = (NCHUNKS_N - wid + NW - 1) // NW
        pltpu.async_copy(ones_hbm, ones_v, isem)
        pltpu.async_copy(
            zeros_hbm, sums_sh.at[pl.ds(s * GSTRIPE, GSTRIPE)], isem)
        pltpu.async_copy(
            zeros_hbm.at[0], cnts_sh.at[pl.ds(s * GSTRIPE, GSTRIPE)], isem)
        for b in range(maxc):
            def stage(b=b):
                pltpu.async_copy(batch3_hbm.at[wid + b * NW], idx_all.at[b],
                                 isem)
                pltpu.async_copy(
                    h_hbm.at[pl.ds((wid + b * NW) * CHUNK, CHUNK)],
                    rings.at[b], isem)
            pl.when(b < my_chunks)(stage)
        pltpu.make_async_copy(ones_hbm, ones_v, isem).wait()
        pltpu.make_async_copy(
            zeros_hbm, sums_sh.at[pl.ds(s * GSTRIPE, GSTRIPE)], isem).wait()
        pltpu.make_async_copy(
            zeros_hbm.at[0], cnts_sh.at[pl.ds(s * GSTRIPE, GSTRIPE)],
            isem).wait()
        for b in range(maxc):
            def wstage(b=b):
                pltpu.make_async_copy(batch3_hbm.at[0], idx_all.at[b],
                                      isem).wait()
                pltpu.make_async_copy(h_hbm.at[pl.ds(0, CHUNK)], rings.at[b],
                                      isem).wait()
            pl.when(b < my_chunks)(wstage)
        plsc.subcore_barrier()
        for b in range(maxc):
            def scatter(b=b):
                pltpu.async_copy(rings.at[b], sums_sh.at[idx_all.at[b]],
                                 ssem, add=True)
                pltpu.async_copy(ones_v, cnts_sh.at[idx_all.at[b]],
                                 ssem, add=True)
            pl.when(b < my_chunks)(scatter)
        for b in range(maxc):
            def wscatter(b=b):
                pltpu.make_async_copy(h_hbm.at[pl.ds(0, CHUNK)], rings.at[b],
                                      ssem).wait()
                pltpu.make_async_copy(ones_hbm, ones_v, ssem).wait()
            pl.when(b < my_chunks)(wscatter)
        plsc.subcore_barrier()
        pltpu.sync_copy(sums_sh.at[pl.ds(s * GSTRIPE, GSTRIPE)],
                        sums_hbm.at[c, s])
        pltpu.sync_copy(cnts_sh.at[pl.ds(s * GSTRIPE, GSTRIPE)],
                        cnts_hbm.at[c, s])

    return pool_kernel


_TCB = 2000


def _tc_first(deg_cols, x, w1):
    d = x.shape[1]
    k = w1.shape[1]

    def body(deg_ref, x_ref, w_ref, dinv_ref, hw_ref):
        deg = deg_ref[:, 0:1] + deg_ref[:, 1:2] + 1.0
        dinv = lax.rsqrt(deg)
        hw = jnp.dot(x_ref[...], w_ref[...], preferred_element_type=_f32)
        dinv_ref[...] = dinv
        hw_ref[...] = hw * dinv

    return pl.pallas_call(
        body,
        grid=(N // _TCB,),
        in_specs=[
            pl.BlockSpec((_TCB, NC), lambda i: (i, 0)),
            pl.BlockSpec((_TCB, d), lambda i: (i, 0)),
            pl.BlockSpec((d, k), lambda i: (0, 0)),
        ],
        out_specs=[
            pl.BlockSpec((_TCB, 1), lambda i: (i, 0)),
            pl.BlockSpec((_TCB, k), lambda i: (i, 0)),
        ],
        out_shape=[
            jax.ShapeDtypeStruct((N, 1), _f32),
            jax.ShapeDtypeStruct((N, k), _f32),
        ],
    )(deg_cols, x, w1)


def _tc_mid(acc, hwp, dinv, b, w_next):
    k = hwp.shape[1]
    k2 = w_next.shape[1]

    def body(acc_ref, hwp_ref, dinv_ref, b_ref, w_ref, out_ref):
        t = acc_ref[0] + acc_ref[1] + hwp_ref[...]
        h = jnp.maximum(t * dinv_ref[...] + b_ref[...], 0.0)
        out_ref[...] = (
            jnp.dot(h, w_ref[...], preferred_element_type=_f32)
            * dinv_ref[...])

    return pl.pallas_call(
        body,
        grid=(N // _TCB,),
        in_specs=[
            pl.BlockSpec((NC, _TCB, k), lambda i: (0, i, 0)),
            pl.BlockSpec((_TCB, k), lambda i: (i, 0)),
            pl.BlockSpec((_TCB, 1), lambda i: (i, 0)),
            pl.BlockSpec((1, k), lambda i: (0, 0)),
            pl.BlockSpec((k, k2), lambda i: (0, 0)),
        ],
        out_specs=pl.BlockSpec((_TCB, k2), lambda i: (i, 0)),
        out_shape=jax.ShapeDtypeStruct((N, k2), _f32),
    )(acc, hwp, dinv, b, w_next)


def _tc_last_h(acc, hwp, dinv, b):
    k = hwp.shape[1]

    def body(acc_ref, hwp_ref, dinv_ref, b_ref, out_ref):
        t = acc_ref[0] + acc_ref[1] + hwp_ref[...]
        out_ref[...] = jnp.maximum(t * dinv_ref[...] + b_ref[...], 0.0)

    return pl.pallas_call(
        body,
        grid=(N // _TCB,),
        in_specs=[
            pl.BlockSpec((NC, _TCB, k), lambda i: (0, i, 0)),
            pl.BlockSpec((_TCB, k), lambda i: (i, 0)),
            pl.BlockSpec((_TCB, 1), lambda i: (i, 0)),
            pl.BlockSpec((1, k), lambda i: (0, 0)),
        ],
        out_specs=pl.BlockSpec((_TCB, k), lambda i: (i, 0)),
        out_shape=jax.ShapeDtypeStruct((N, k), _f32),
    )(acc, hwp, dinv, b)


def _tc_head(sums, cnts_cols, lin_w, lin_b):
    k = sums.shape[2]
    ncls = lin_w.shape[1]

    def body(s_ref, c_ref, w_ref, b_ref, out_ref):
        cnt = c_ref[:, 0:1] + c_ref[:, 1:2]
        pooled = (s_ref[0] + s_ref[1]) / jnp.maximum(cnt, 1.0)
        out_ref[...] = (
            jnp.dot(pooled, w_ref[...], preferred_element_type=_f32)
            + b_ref[...])

    return pl.pallas_call(
        body,
        grid=(1,),
        in_specs=[
            pl.BlockSpec((NC, G, k), lambda i: (0, 0, 0)),
            pl.BlockSpec((G, NC), lambda i: (0, 0)),
            pl.BlockSpec((k, ncls), lambda i: (0, 0)),
            pl.BlockSpec((1, ncls), lambda i: (0, 0)),
        ],
        out_specs=pl.BlockSpec((G, ncls), lambda i: (0, 0)),
        out_shape=jax.ShapeDtypeStruct((G, ncls), _f32),
    )(sums, cnts_cols, lin_w, lin_b)


_deg_kernel = _make_deg_kernel()
_edge64 = _make_edge_kernel(64, 80)
_edge128 = _make_edge_kernel(128, 40)
_pool64 = _make_pool_kernel(64)


def kernel(x, edge_index, batch, W1, b1, W2, b2, W3, b3, lin_W, lin_b):
    src3 = edge_index[0].reshape(NW, NCHUNKS_E, CHUNK)
    dst3 = edge_index[1].reshape(NW, NCHUNKS_E, CHUNK)
    src3n = edge_index[0].reshape(NW, E // NW // 40, 40)
    dst3n = edge_index[1].reshape(NW, E // NW // 40, 40)
    batch3 = batch.reshape(NCHUNKS_N, CHUNK)

    ones80 = jnp.ones((CHUNK,), _f32)
    zeros1d = jnp.zeros((STRIPE,), _f32)
    zeros64 = jnp.zeros((STRIPE, 64), _f32)
    zeros128 = jnp.zeros((STRIPE, 128), _f32)
    zeros_g = jnp.zeros((GSTRIPE, 64), _f32)

    deg_p = _deg_kernel(dst3, ones80, zeros1d)
    deg_cols = deg_p.reshape(NC, NPAD).T
    dinv, hw1p = _tc_first(deg_cols, x, W1)

    acc1 = _edge64(hw1p, src3, dst3, zeros64).reshape(NC, NPAD, 64)
    hw2p = _tc_mid(acc1, hw1p, dinv, b1.reshape(1, -1), W2)

    acc2 = _edge128(hw2p, src3n, dst3n, zeros128).reshape(NC, NPAD, 128)
    hw3p = _tc_mid(acc2, hw2p, dinv, b2.reshape(1, -1), W3)

    acc3 = _edge64(hw3p, src3, dst3, zeros64).reshape(NC, NPAD, 64)
    h3 = _tc_last_h(acc3, hw3p, dinv, b3.reshape(1, -1))

    sums, cnts = _pool64(h3, batch3, ones80, zeros_g)
    out = _tc_head(sums.reshape(NC, GPAD, 64),
                   cnts.reshape(NC, GPAD).T,
                   lin_W, lin_b.reshape(1, -1))
    return out
